# baseline (speedup 1.0000x reference)
import functools

import jax
import jax.numpy as jnp
from jax import lax
from jax.experimental import pallas as pl
from jax.experimental.pallas import tpu as pltpu
from jax.experimental.pallas import tpu_sc as plsc

_LANES = 16
_CH = 80
_STAGE = 40
_NB = 6
_LOOK = 3


@functools.lru_cache(maxsize=None)
def _make_kernel(n_nodes, n_edges, n_rows, d):
    info = plsc.get_sparse_core_info()
    nc, ns = info.num_cores, info.num_subcores
    nw = nc * ns
    assert n_edges % nw == 0
    epw = n_edges // nw
    assert epw % _CH == 0
    nch = epw // _CH
    assert nch > 2 * _NB
    assert n_rows % _STAGE == 0 and _STAGE <= _CH
    n_main = nch - 2 * _LOOK
    n_fori = n_main // _NB
    n_tail = n_main - n_fori * _NB

    mesh = plsc.VectorSubcoreMesh(core_axis_name="c", subcore_axis_name="s")

    @functools.partial(
        pl.kernel,
        out_type=jax.ShapeDtypeStruct((n_edges, d), jnp.float32),
        mesh=mesh,
        scratch_types=[
            pltpu.VMEM((n_nodes,), jnp.int32),
            pltpu.VMEM((epw,), jnp.int32),
            pltpu.VMEM((epw,), jnp.int32),
            pltpu.VMEM((epw,), jnp.int32),
            [pltpu.VMEM((_CH, d), jnp.float32)] * _NB,
            pltpu.VMEM_SHARED((n_rows, d), jnp.float32),
            [pltpu.SemaphoreType.DMA] * _NB,
            [pltpu.SemaphoreType.DMA] * _NB,
        ],
        compiler_params=pltpu.CompilerParams(needs_layout_passes=False),
    )
    def edge_emb(nt_hbm, ei_hbm, tab_hbm, out_hbm,
                 nt_v, src_v, dst_v, et_v, bufs, tab_sh, sem_g, sem_o):
        sid = lax.axis_index("s")
        wid = sid * nc + lax.axis_index("c")
        base = wid * epw

        for ch in range(n_rows // _STAGE):
            @pl.when(sid == (ch % ns))
            def _stage_chunk(ch=ch):
                r0 = ch * _STAGE
                pltpu.sync_copy(tab_hbm.at[pl.ds(r0, _STAGE)],
                                bufs[0].at[pl.ds(0, _STAGE)])
                pltpu.sync_copy(bufs[0].at[pl.ds(0, _STAGE)],
                                tab_sh.at[pl.ds(r0, _STAGE)])

        pltpu.sync_copy(nt_hbm, nt_v)
        pltpu.sync_copy(ei_hbm.at[pl.ds(base, epw)], src_v)
        pltpu.sync_copy(ei_hbm.at[pl.ds(n_edges + base, epw)], dst_v)

        def compute_chunk(j):
            for k in range(_CH // _LANES):
                off = j * _CH + k * _LANES
                sv = src_v[pl.ds(off, _LANES)]
                dv = dst_v[pl.ds(off, _LANES)]
                ax = plsc.load_gather(nt_v, [sv])
                ay = plsc.load_gather(nt_v, [dv])
                dd = jnp.abs(ax - ay) - 1
                et_v[pl.ds(off, _LANES)] = ax * ay + ((dd * dd) >> 2)

        def g_copy(j, b):
            tab = tab_sh if b % 2 == 0 else tab_hbm
            return pltpu.make_async_copy(
                tab.at[et_v.at[pl.ds(j * _CH, _CH)]], bufs[b], sem_g[b])

        def w_copy(j, b):
            return pltpu.make_async_copy(
                bufs[b], out_hbm.at[pl.ds(base + j * _CH, _CH)], sem_o[b])

        for j in range(_LOOK):
            compute_chunk(j)
        plsc.subcore_barrier()
        for j in range(_LOOK):
            g_copy(j, j).start()
        for j in range(_LOOK):
            g_copy(j, j).wait()
            w_copy(j, j).start()
            compute_chunk(j + _LOOK)
            g_copy(j + _LOOK, j + _LOOK).start()

        def step(j, b, bn):
            g_copy(j, b).wait()
            w_copy(j, b).start()
            compute_chunk(j + _LOOK)
            w_copy(j - _LOOK, bn).wait()
            g_copy(j + _LOOK, bn).start()

        def loop(j0, carry):
            j = _LOOK + j0 * _NB
            for u in range(_NB):
                bu = (_LOOK + u) % _NB
                step(j + u, bu, (bu + _LOOK) % _NB)
            return carry

        lax.fori_loop(0, n_fori, loop, 0)
        jt = _LOOK + n_fori * _NB
        for u in range(n_tail):
            j = jt + u
            b = j % _NB
            step(j, b, (b + _LOOK) % _NB)

        for j in range(nch - _LOOK, nch):
            b = j % _NB
            g_copy(j, b).wait()
            w_copy(j, b).start()
        for j in range(nch - 2 * _LOOK, nch):
            w_copy(j, j % _NB).wait()

    return edge_emb


def kernel(node_type, edge_index, emb_table):
    (n_nodes,) = node_type.shape
    _, n_edges = edge_index.shape
    n_rows, d = emb_table.shape
    f = _make_kernel(n_nodes, n_edges, n_rows, d)
    return f(node_type.astype(jnp.int32), jnp.ravel(edge_index), emb_table)


# speedup vs baseline: 1.4501x; 1.4501x over previous
import functools

import jax
import jax.numpy as jnp
from jax import lax
from jax.experimental import pallas as pl
from jax.experimental.pallas import tpu as pltpu
from jax.experimental.pallas import tpu_sc as plsc

_LANES = 16
_CH = 80
_STAGE = 40
_NB = 6
_LOOK = 3


@functools.lru_cache(maxsize=None)
def _make_kernel(n_nodes, n_edges, n_rows, d):
    info = plsc.get_sparse_core_info()
    nc, ns = info.num_cores, info.num_subcores
    nw = nc * ns
    assert n_edges % nw == 0
    epw = n_edges // nw
    assert epw % _CH == 0
    nch = epw // _CH
    assert nch > 2 * _NB
    assert n_rows % _STAGE == 0 and _STAGE <= _CH
    n_main = nch - 2 * _LOOK
    n_fori = n_main // _NB
    n_tail = n_main - n_fori * _NB

    mesh = plsc.VectorSubcoreMesh(core_axis_name="c", subcore_axis_name="s")

    @functools.partial(
        pl.kernel,
        out_type=jax.ShapeDtypeStruct((n_edges, d), jnp.float32),
        mesh=mesh,
        scratch_types=[
            pltpu.VMEM((n_nodes,), jnp.int32),
            pltpu.VMEM((epw,), jnp.int32),
            pltpu.VMEM((epw,), jnp.int32),
            pltpu.VMEM((epw,), jnp.int32),
            [pltpu.VMEM((_CH, d), jnp.float32)] * _NB,
            pltpu.VMEM_SHARED((n_rows, d), jnp.float32),
            [pltpu.SemaphoreType.DMA] * _NB,
            [pltpu.SemaphoreType.DMA] * _NB,
        ],
        compiler_params=pltpu.CompilerParams(
            needs_layout_passes=False,
            disable_bounds_checks=True,
            disable_semaphore_checks=True,
        ),
    )
    def edge_emb(nt_hbm, ei_hbm, tab_hbm, out_hbm,
                 nt_v, src_v, dst_v, et_v, bufs, tab_sh, sem_g, sem_o):
        sid = lax.axis_index("s")
        wid = sid * nc + lax.axis_index("c")
        base = wid * epw

        for ch in range(n_rows // _STAGE):
            @pl.when(sid == (ch % ns))
            def _stage_chunk(ch=ch):
                r0 = ch * _STAGE
                pltpu.sync_copy(tab_hbm.at[pl.ds(r0, _STAGE)],
                                bufs[0].at[pl.ds(0, _STAGE)])
                pltpu.sync_copy(bufs[0].at[pl.ds(0, _STAGE)],
                                tab_sh.at[pl.ds(r0, _STAGE)])

        pltpu.sync_copy(nt_hbm, nt_v)
        pltpu.sync_copy(ei_hbm.at[pl.ds(base, epw)], src_v)
        pltpu.sync_copy(ei_hbm.at[pl.ds(n_edges + base, epw)], dst_v)

        def compute_chunk(j):
            for k in range(_CH // _LANES):
                off = j * _CH + k * _LANES
                sv = src_v[pl.ds(off, _LANES)]
                dv = dst_v[pl.ds(off, _LANES)]
                ax = plsc.load_gather(nt_v, [sv])
                ay = plsc.load_gather(nt_v, [dv])
                dd = jnp.abs(ax - ay) - 1
                et_v[pl.ds(off, _LANES)] = ax * ay + ((dd * dd) >> 2)

        def g_copy(j, b):
            return pltpu.make_async_copy(
                tab_sh.at[et_v.at[pl.ds(j * _CH, _CH)]], bufs[b], sem_g[b])

        def w_copy(j, b):
            return pltpu.make_async_copy(
                bufs[b], out_hbm.at[pl.ds(base + j * _CH, _CH)], sem_o[b])

        for j in range(_LOOK):
            compute_chunk(j)
        plsc.subcore_barrier()
        for j in range(_LOOK):
            g_copy(j, j).start()
        for j in range(_LOOK):
            g_copy(j, j).wait()
            w_copy(j, j).start()
            compute_chunk(j + _LOOK)
            g_copy(j + _LOOK, j + _LOOK).start()

        def step(j, b, bn):
            g_copy(j, b).wait()
            w_copy(j, b).start()
            compute_chunk(j + _LOOK)
            w_copy(j - _LOOK, bn).wait()
            g_copy(j + _LOOK, bn).start()

        def loop(j0, carry):
            j = _LOOK + j0 * _NB
            for u in range(_NB):
                bu = (_LOOK + u) % _NB
                step(j + u, bu, (bu + _LOOK) % _NB)
            return carry

        lax.fori_loop(0, n_fori, loop, 0)
        jt = _LOOK + n_fori * _NB
        for u in range(n_tail):
            j = jt + u
            b = j % _NB
            step(j, b, (b + _LOOK) % _NB)

        for j in range(nch - _LOOK, nch):
            b = j % _NB
            g_copy(j, b).wait()
            w_copy(j, b).start()
        for j in range(nch - 2 * _LOOK, nch):
            w_copy(j, j % _NB).wait()

    return edge_emb


def kernel(node_type, edge_index, emb_table):
    (n_nodes,) = node_type.shape
    _, n_edges = edge_index.shape
    n_rows, d = emb_table.shape
    f = _make_kernel(n_nodes, n_edges, n_rows, d)
    return f(node_type.astype(jnp.int32), jnp.ravel(edge_index), emb_table)


# speedup vs baseline: 1.4657x; 1.0107x over previous
import functools

import jax
import jax.numpy as jnp
from jax import lax
from jax.experimental import pallas as pl
from jax.experimental.pallas import tpu as pltpu
from jax.experimental.pallas import tpu_sc as plsc

_LANES = 16
_CH = 80
_STAGE = 40
_NB = 6
_LOOK = 3


@functools.lru_cache(maxsize=None)
def _make_kernel(n_nodes, n_edges, n_rows, d):
    info = plsc.get_sparse_core_info()
    nc, ns = info.num_cores, info.num_subcores
    nw = nc * ns
    assert n_edges % nw == 0
    epw = n_edges // nw
    assert epw % _CH == 0
    nch = epw // _CH
    assert nch > 2 * _NB
    assert n_rows % _STAGE == 0 and _STAGE <= _CH
    n_main = nch - 2 * _LOOK
    n_fori = n_main // _NB
    n_tail = n_main - n_fori * _NB

    mesh = plsc.VectorSubcoreMesh(core_axis_name="c", subcore_axis_name="s")

    @functools.partial(
        pl.kernel,
        out_type=jax.ShapeDtypeStruct((n_edges, d), jnp.float32),
        mesh=mesh,
        scratch_types=[
            pltpu.VMEM((n_nodes,), jnp.int32),
            pltpu.VMEM((epw,), jnp.int32),
            pltpu.VMEM((epw,), jnp.int32),
            pltpu.VMEM((epw,), jnp.int32),
            [pltpu.VMEM((_CH, d), jnp.float32)] * _NB,
            pltpu.VMEM_SHARED((n_rows, d), jnp.float32),
            [pltpu.SemaphoreType.DMA] * _NB,
            [pltpu.SemaphoreType.DMA] * _NB,
        ],
        compiler_params=pltpu.CompilerParams(needs_layout_passes=False),
    )
    def edge_emb(nt_hbm, ei_hbm, tab_hbm, out_hbm,
                 nt_v, src_v, dst_v, et_v, bufs, tab_sh, sem_g, sem_o):
        sid = lax.axis_index("s")
        wid = sid * nc + lax.axis_index("c")
        base = wid * epw

        in_cps = (
            pltpu.make_async_copy(nt_hbm, nt_v, sem_g[0]),
            pltpu.make_async_copy(ei_hbm.at[pl.ds(base, epw)], src_v,
                                  sem_g[1]),
            pltpu.make_async_copy(ei_hbm.at[pl.ds(n_edges + base, epw)],
                                  dst_v, sem_g[2]),
        )
        for cp in in_cps:
            cp.start()

        for ch in range(n_rows // _STAGE):
            @pl.when(sid == (ch % ns))
            def _stage_chunk(ch=ch):
                r0 = ch * _STAGE
                pltpu.sync_copy(tab_hbm.at[pl.ds(r0, _STAGE)],
                                bufs[0].at[pl.ds(0, _STAGE)])
                pltpu.sync_copy(bufs[0].at[pl.ds(0, _STAGE)],
                                tab_sh.at[pl.ds(r0, _STAGE)])

        for cp in in_cps:
            cp.wait()

        def compute_chunk(j):
            for k in range(_CH // _LANES):
                off = j * _CH + k * _LANES
                sv = src_v[pl.ds(off, _LANES)]
                dv = dst_v[pl.ds(off, _LANES)]
                ax = plsc.load_gather(nt_v, [sv])
                ay = plsc.load_gather(nt_v, [dv])
                dd = jnp.abs(ax - ay) - 1
                et_v[pl.ds(off, _LANES)] = ax * ay + ((dd * dd) >> 2)

        def g_copy(j, b):
            return pltpu.make_async_copy(
                tab_sh.at[et_v.at[pl.ds(j * _CH, _CH)]], bufs[b], sem_g[b])

        def w_copy(j, b):
            return pltpu.make_async_copy(
                bufs[b], out_hbm.at[pl.ds(base + j * _CH, _CH)], sem_o[b])

        for j in range(_LOOK):
            compute_chunk(j)
        plsc.subcore_barrier()
        for j in range(_LOOK):
            g_copy(j, j).start()
        for j in range(_LOOK):
            g_copy(j, j).wait()
            w_copy(j, j).start()
            compute_chunk(j + _LOOK)
            g_copy(j + _LOOK, j + _LOOK).start()

        def step(j, b, bn):
            g_copy(j, b).wait()
            w_copy(j, b).start()
            compute_chunk(j + _LOOK)
            w_copy(j - _LOOK, bn).wait()
            g_copy(j + _LOOK, bn).start()

        def loop(j0, carry):
            j = _LOOK + j0 * _NB
            for u in range(_NB):
                bu = (_LOOK + u) % _NB
                step(j + u, bu, (bu + _LOOK) % _NB)
            return carry

        lax.fori_loop(0, n_fori, loop, 0)
        jt = _LOOK + n_fori * _NB
        for u in range(n_tail):
            j = jt + u
            b = j % _NB
            step(j, b, (b + _LOOK) % _NB)

        for j in range(nch - _LOOK, nch):
            b = j % _NB
            g_copy(j, b).wait()
            w_copy(j, b).start()
        for j in range(nch - 2 * _LOOK, nch):
            w_copy(j, j % _NB).wait()

    return edge_emb


def kernel(node_type, edge_index, emb_table):
    (n_nodes,) = node_type.shape
    _, n_edges = edge_index.shape
    n_rows, d = emb_table.shape
    f = _make_kernel(n_nodes, n_edges, n_rows, d)
    return f(node_type.astype(jnp.int32), jnp.ravel(edge_index), emb_table)
